# baseline (speedup 1.0000x reference)
import functools

import jax
import jax.numpy as jnp
from jax import lax
from jax.experimental import pallas as pl
from jax.experimental.pallas import tpu as pltpu
from jax.experimental.pallas import tpu_sc as plsc

EMB = 64
HID = 64
NC, NS = 2, 16
NW = NC * NS
CHUNK = 128


def _sc_gather(idx3d, table):
    cpw = idx3d.shape[1]
    n_rows = NW * cpw * CHUNK

    mesh = plsc.VectorSubcoreMesh(core_axis_name="c", subcore_axis_name="s")

    @functools.partial(
        pl.kernel,
        out_type=jax.ShapeDtypeStruct((n_rows, EMB), jnp.float32),
        mesh=mesh,
        scratch_types=[
            pltpu.VMEM((cpw, CHUNK), jnp.int32),
            pltpu.VMEM((CHUNK, EMB), jnp.float32),
            pltpu.SemaphoreType.DMA,
        ],
        compiler_params=pltpu.CompilerParams(use_tc_tiling_on_sc=False),
    )
    def k(idx_hbm, table_hbm, out_hbm, idx_v, rows_v, gsem):
        wid = lax.axis_index("s") * NC + lax.axis_index("c")
        c0 = wid * cpw
        pltpu.sync_copy(idx_hbm.at[wid], idx_v)

        def body(g, carry):
            pltpu.async_copy(table_hbm.at[idx_v.at[g]], rows_v, gsem).wait()
            pltpu.sync_copy(rows_v, out_hbm.at[pl.ds((c0 + g) * CHUNK, CHUNK)])
            return carry

        lax.fori_loop(0, cpw, body, 0)

    return k(idx3d, table)


def _tc_project(gathered, W, half_rows):
    B = 2048
    K = half_rows // B

    def body(x1_ref, x2_ref, w_ref, o1_ref, o2_ref):
        w = w_ref[...]
        dn = (((1,), (1,)), ((), ()))
        o1_ref[...] = lax.dot_general(x1_ref[...], w, dn,
                                      preferred_element_type=jnp.float32)
        o2_ref[...] = lax.dot_general(x2_ref[...], w, dn,
                                      preferred_element_type=jnp.float32)

    return pl.pallas_call(
        body,
        grid=(K,),
        in_specs=[
            pl.BlockSpec((B, EMB), lambda i: (i, 0)),
            pl.BlockSpec((B, EMB), lambda i, _K=K: (i + _K, 0)),
            pl.BlockSpec((HID, EMB), lambda i: (0, 0)),
        ],
        out_specs=[
            pl.BlockSpec((B, HID), lambda i: (i, 0)),
            pl.BlockSpec((B, HID), lambda i: (i, 0)),
        ],
        out_shape=[jax.ShapeDtypeStruct((half_rows, HID), jnp.float32)] * 2,
    )(gathered, gathered, W)


def kernel(sent1, sent2, embedding, W):
    batch, seq = sent1.shape
    half_rows = batch * seq
    idx = jnp.concatenate(
        [sent1.reshape(-1), sent2.reshape(-1)]).astype(jnp.int32)
    idx3d = idx.reshape(NW, -1, CHUNK)
    gathered = _sc_gather(idx3d, embedding)
    o1, o2 = _tc_project(gathered, W, half_rows)
    return (o1.reshape(batch, seq, HID), o2.reshape(batch, seq, HID))


# speedup vs baseline: 1.3426x; 1.3426x over previous
import functools

import jax
import jax.numpy as jnp
from jax import lax
from jax.experimental import pallas as pl
from jax.experimental.pallas import tpu as pltpu
from jax.experimental.pallas import tpu_sc as plsc

EMB = 64
HID = 64
NC, NS = 2, 16
NW = NC * NS
CHUNK = 128
PBLK = 4096


def _tc_project_table(table_t, W):
    V = table_t.shape[1]
    grid = (V + PBLK - 1) // PBLK

    def body(et_ref, w_ref, p_ref):
        y = lax.dot_general(et_ref[...], w_ref[...], (((0,), (1,)), ((), ())),
                            preferred_element_type=jnp.float32)
        p_ref[...] = jnp.concatenate(
            [y, jnp.zeros((PBLK, 128 - HID), jnp.float32)], axis=1)

    return pl.pallas_call(
        body,
        grid=(grid,),
        in_specs=[
            pl.BlockSpec((EMB, PBLK), lambda i: (0, i)),
            pl.BlockSpec((HID, EMB), lambda i: (0, 0)),
        ],
        out_specs=pl.BlockSpec((PBLK, 128), lambda i: (i, 0)),
        out_shape=jax.ShapeDtypeStruct((V, 128), jnp.float32),
    )(table_t, W)


def _sc_gather(idx3d, table):
    cpw = idx3d.shape[1]
    n_rows = NW * cpw * CHUNK

    mesh = plsc.VectorSubcoreMesh(core_axis_name="c", subcore_axis_name="s")

    @functools.partial(
        pl.kernel,
        out_type=jax.ShapeDtypeStruct((n_rows, 128), jnp.float32),
        mesh=mesh,
        scratch_types=[
            pltpu.VMEM((cpw, CHUNK), jnp.int32),
            pltpu.VMEM((CHUNK, 128), jnp.float32),
            pltpu.SemaphoreType.DMA,
        ],
    )
    def k(idx_hbm, table_hbm, out_hbm, idx_v, rows_v, gsem):
        wid = lax.axis_index("s") * NC + lax.axis_index("c")
        c0 = wid * cpw
        pltpu.sync_copy(idx_hbm.at[wid], idx_v)

        def body(g, carry):
            pltpu.async_copy(table_hbm.at[idx_v.at[g]], rows_v, gsem).wait()
            pltpu.sync_copy(rows_v, out_hbm.at[pl.ds((c0 + g) * CHUNK, CHUNK)])
            return carry

        lax.fori_loop(0, cpw, body, 0)

    return k(idx3d, table)


def kernel(sent1, sent2, embedding, W):
    batch, seq = sent1.shape
    half_rows = batch * seq
    proj = _tc_project_table(embedding.T, W)
    idx = jnp.concatenate(
        [sent1.reshape(-1), sent2.reshape(-1)]).astype(jnp.int32)
    idx3d = idx.reshape(NW, -1, CHUNK)
    g = _sc_gather(idx3d, proj)
    o1 = g[:half_rows, :HID].reshape(batch, seq, HID)
    o2 = g[half_rows:, :HID].reshape(batch, seq, HID)
    return (o1, o2)


# speedup vs baseline: 1.6809x; 1.2519x over previous
import functools

import jax
import jax.numpy as jnp
from jax import lax
from jax.experimental import pallas as pl
from jax.experimental.pallas import tpu as pltpu
from jax.experimental.pallas import tpu_sc as plsc

EMB = 64
HID = 64
NC, NS = 2, 16
NW = NC * NS
CHUNK = 128
PBLK = 4096


def _tc_project_table(table_t, W):
    V = table_t.shape[1]
    grid = (V + PBLK - 1) // PBLK

    def body(et_ref, w_ref, p_ref):
        y = lax.dot_general(et_ref[...], w_ref[...], (((0,), (1,)), ((), ())),
                            preferred_element_type=jnp.float32)
        p_ref[...] = jnp.concatenate(
            [y, jnp.zeros((PBLK, 128 - HID), jnp.float32)], axis=1)

    return pl.pallas_call(
        body,
        grid=(grid,),
        in_specs=[
            pl.BlockSpec((EMB, PBLK), lambda i: (0, i)),
            pl.BlockSpec((HID, EMB), lambda i: (0, 0)),
        ],
        out_specs=pl.BlockSpec((PBLK, 128), lambda i: (i, 0)),
        out_shape=jax.ShapeDtypeStruct((V, 128), jnp.float32),
    )(table_t, W)


def _sc_gather(idx1, idx2, table):
    hw = NW // 2
    cpw = idx1.shape[1]
    half_rows = hw * cpw * CHUNK

    mesh = plsc.VectorSubcoreMesh(core_axis_name="c", subcore_axis_name="s")

    @functools.partial(
        pl.kernel,
        out_type=(jax.ShapeDtypeStruct((half_rows, 128), jnp.float32),
                  jax.ShapeDtypeStruct((half_rows, 128), jnp.float32)),
        mesh=mesh,
        scratch_types=[
            pltpu.VMEM((cpw, CHUNK), jnp.int32),
            pltpu.VMEM((2, CHUNK, 128), jnp.float32),
            pltpu.SemaphoreType.DMA,
            pltpu.SemaphoreType.DMA,
            pltpu.SemaphoreType.DMA,
            pltpu.SemaphoreType.DMA,
        ],
    )
    def k(idx1_hbm, idx2_hbm, table_hbm, out1_hbm, out2_hbm,
          idx_v, rows_v, gs0, gs1, os0, os1):
        wid = lax.axis_index("s") * NC + lax.axis_index("c")
        sid = wid // hw
        ww = wid % hw
        gsems = (gs0, gs1)
        osems = (os0, os1)

        def run(idx_hbm, out_hbm):
            pltpu.sync_copy(idx_hbm.at[ww], idx_v)

            def gather(g, b):
                return pltpu.make_async_copy(
                    table_hbm.at[idx_v.at[g]], rows_v.at[b], gsems[b])

            def scatter(g, b):
                return pltpu.make_async_copy(
                    rows_v.at[b],
                    out_hbm.at[pl.ds((ww * cpw + g) * CHUNK, CHUNK)],
                    osems[b])

            gather(0, 0).start()

            @pl.loop(0, cpw, step=2)
            def _(i):
                for b in (0, 1):
                    g = i + b
                    @pl.when(g + 1 < cpw)
                    def _():
                        @pl.when(g >= 1)
                        def _():
                            scatter(g - 1, 1 - b).wait()
                        gather(g + 1, 1 - b).start()

                    gather(g, b).wait()
                    scatter(g, b).start()

            scatter(cpw - 2, 0).wait()
            scatter(cpw - 1, 1).wait()

        @pl.when(sid == 0)
        def _():
            run(idx1_hbm, out1_hbm)

        @pl.when(sid == 1)
        def _():
            run(idx2_hbm, out2_hbm)

    return k(idx1, idx2, table)


def kernel(sent1, sent2, embedding, W):
    batch, seq = sent1.shape
    hw = NW // 2
    proj = _tc_project_table(embedding.T, W)
    idx1 = sent1.reshape(hw, -1, CHUNK).astype(jnp.int32)
    idx2 = sent2.reshape(hw, -1, CHUNK).astype(jnp.int32)
    g1, g2 = _sc_gather(idx1, idx2, proj)
    o1 = g1[:, :HID].reshape(batch, seq, HID)
    o2 = g2[:, :HID].reshape(batch, seq, HID)
    return (o1, o2)


# speedup vs baseline: 2.1560x; 1.2826x over previous
import functools

import jax
import jax.numpy as jnp
from jax import lax
from jax.experimental import pallas as pl
from jax.experimental.pallas import tpu as pltpu
from jax.experimental.pallas import tpu_sc as plsc

EMB = 64
HID = 64
NC, NS = 2, 16
NW = NC * NS
CHUNK = 128
PBLK = 4096


def _tc_project_table(table_t, W):
    V = table_t.shape[1]
    grid = (V + PBLK - 1) // PBLK

    def body(et_ref, w_ref, p_ref):
        y = lax.dot_general(et_ref[...], w_ref[...], (((0,), (1,)), ((), ())),
                            preferred_element_type=jnp.float32)
        p_ref[...] = jnp.concatenate(
            [y, jnp.zeros((PBLK, 128 - HID), jnp.float32)], axis=1)

    return pl.pallas_call(
        body,
        grid=(grid,),
        in_specs=[
            pl.BlockSpec((EMB, PBLK), lambda i: (0, i)),
            pl.BlockSpec((HID, EMB), lambda i: (0, 0)),
        ],
        out_specs=pl.BlockSpec((PBLK, 128), lambda i: (i, 0)),
        out_shape=jax.ShapeDtypeStruct((V, 128), jnp.float32),
    )(table_t, W)


def _sc_gather(idx1, idx2, table):
    hw = NW // 2
    cpw = idx1.shape[1]
    half_rows = hw * cpw * CHUNK

    mesh = plsc.VectorSubcoreMesh(core_axis_name="c", subcore_axis_name="s")

    @functools.partial(
        pl.kernel,
        out_type=(jax.ShapeDtypeStruct((half_rows, 128), jnp.float32),
                  jax.ShapeDtypeStruct((half_rows, 128), jnp.float32)),
        mesh=mesh,
        scratch_types=[
            pltpu.VMEM((cpw, CHUNK), jnp.int32),
            pltpu.VMEM((2, CHUNK, 128), jnp.float32),
            pltpu.SemaphoreType.DMA,
            pltpu.SemaphoreType.DMA,
            pltpu.SemaphoreType.DMA,
            pltpu.SemaphoreType.DMA,
        ],
    )
    def k(idx1_hbm, idx2_hbm, table_hbm, out1_hbm, out2_hbm,
          idx_v, rows_v, gs0, gs1, os0, os1):
        wid = lax.axis_index("s") * NC + lax.axis_index("c")
        sid = wid // hw
        ww = wid % hw
        gsems = (gs0, gs1)
        osems = (os0, os1)

        def run(idx_hbm, out_hbm):
            pltpu.sync_copy(idx_hbm.at[ww], idx_v)

            def gather(g, b):
                return pltpu.make_async_copy(
                    table_hbm.at[idx_v.at[g]], rows_v.at[b], gsems[b])

            def scatter(g, b):
                return pltpu.make_async_copy(
                    rows_v.at[b],
                    out_hbm.at[pl.ds((ww * cpw + g) * CHUNK, CHUNK)],
                    osems[b])

            gather(0, 0).start()

            @pl.loop(0, cpw, step=2)
            def _(i):
                for b in (0, 1):
                    g = i + b
                    @pl.when(g + 1 < cpw)
                    def _():
                        @pl.when(g >= 1)
                        def _():
                            scatter(g - 1, 1 - b).wait()
                        gather(g + 1, 1 - b).start()

                    gather(g, b).wait()
                    scatter(g, b).start()

            scatter(cpw - 2, 0).wait()
            scatter(cpw - 1, 1).wait()

        @pl.when(sid == 0)
        def _():
            run(idx1_hbm, out1_hbm)

        @pl.when(sid == 1)
        def _():
            run(idx2_hbm, out2_hbm)

    return k(idx1, idx2, table)


def _tc_finalize(g1, g2, batch, seq):
    def body(x1_ref, x2_ref, o1_ref, o2_ref):
        o1_ref[...] = jnp.transpose(x1_ref[...])[None]
        o2_ref[...] = jnp.transpose(x2_ref[...])[None]

    return pl.pallas_call(
        body,
        grid=(seq,),
        in_specs=[
            pl.BlockSpec((batch, HID), lambda i: (i, 0)),
            pl.BlockSpec((batch, HID), lambda i: (i, 0)),
        ],
        out_specs=[
            pl.BlockSpec((1, HID, batch), lambda i: (i, 0, 0)),
            pl.BlockSpec((1, HID, batch), lambda i: (i, 0, 0)),
        ],
        out_shape=[jax.ShapeDtypeStruct((seq, HID, batch), jnp.float32)] * 2,
    )(g1, g2)


def kernel(sent1, sent2, embedding, W):
    batch, seq = sent1.shape
    hw = NW // 2
    proj = _tc_project_table(embedding.T, W)
    idx1 = sent1.T.reshape(hw, -1, CHUNK).astype(jnp.int32)
    idx2 = sent2.T.reshape(hw, -1, CHUNK).astype(jnp.int32)
    g1, g2 = _sc_gather(idx1, idx2, proj)
    t1, t2 = _tc_finalize(g1[:, :HID], g2[:, :HID], batch, seq)
    return (jnp.transpose(t1, (2, 0, 1)), jnp.transpose(t2, (2, 0, 1)))
